# baseline (speedup 1.0000x reference)
import functools
import jax
import jax.numpy as jnp
from jax import lax
from jax.experimental import pallas as pl
from jax.experimental.pallas import tpu as pltpu
from jax.experimental.pallas import tpu_sc as plsc

D = 64
SCALE = 8.0
L = 16

_info = plsc.get_sparse_core_info()
NC, NS = _info.num_cores, _info.num_subcores
NW = NC * NS

BATCH = 128
NBUF = 2


def _make_emb(B):
    assert B % (NW * BATCH * NBUF) == 0
    b_per_w = B // NW
    n_batches = b_per_w // BATCH
    mesh = plsc.VectorSubcoreMesh(core_axis_name="c", subcore_axis_name="s")

    @functools.partial(
        pl.kernel, mesh=mesh,
        out_type=jax.ShapeDtypeStruct((B, D), jnp.float32),
        compiler_params=pltpu.CompilerParams(needs_layout_passes=False),
        scratch_types=[
            pltpu.VMEM((b_per_w,), jnp.int32),
            pltpu.VMEM((n_batches, BATCH), jnp.int32),
            pltpu.VMEM((NBUF, BATCH, 2 * D), jnp.float32),
            pltpu.VMEM((NBUF, BATCH, D), jnp.float32),
            pltpu.SemaphoreType.DMA,
            pltpu.SemaphoreType.DMA,
            pltpu.SemaphoreType.DMA,
            pltpu.SemaphoreType.DMA,
        ],
    )
    def _emb(idx_hbm, tab_hbm, out_hbm, idx_v, pidx_v, rows_v, outs_v,
             g0, g1, o0, o1):
        gsem = [g0, g1]
        osem = [o0, o1]
        wid = lax.axis_index("s") * NC + lax.axis_index("c")
        base = wid * b_per_w
        pltpu.sync_copy(idx_hbm.at[pl.ds(base, b_per_w)], idx_v)

        def mk_pairs(i, _):
            bq = i // (BATCH // L)
            lq = i % (BATCH // L)
            v = idx_v[pl.ds(i * L, L)]
            pidx_v[bq, pl.ds(lq * L, L)] = jax.lax.shift_right_logical(v, 1)
            return 0

        lax.fori_loop(0, b_per_w // L, mk_pairs, 0)

        def g_desc(bt, b):
            return pltpu.make_async_copy(
                tab_hbm.at[pidx_v.at[bt]], rows_v.at[b], gsem[b])

        def o_desc(bt, b):
            return pltpu.make_async_copy(
                outs_v.at[b], out_hbm.at[pl.ds(base + bt * BATCH, BATCH)],
                osem[b])

        lanes = lax.broadcasted_iota(jnp.int32, (L,), 0)

        def select_scale(bt, b):
            def row(j, _):
                hv = plsc.load_gather(
                    idx_v, [jnp.full((L,), bt * BATCH + j, jnp.int32)])
                off = jnp.bitwise_and(hv, 1) * D + lanes
                for c in range(D // L):
                    vals = plsc.load_gather(
                        rows_v, [jnp.full((L,), b, jnp.int32),
                                 jnp.full((L,), j, jnp.int32),
                                 off + c * L])
                    outs_v[b, j, pl.ds(c * L, L)] = vals * SCALE
                return 0
            lax.fori_loop(0, BATCH, row, 0)

        g_desc(0, 0).start()

        def outer(o, _):
            for b in range(NBUF):
                bt = o * NBUF + b
                nb = (b + 1) % NBUF
                @pl.when(bt + 1 < n_batches)
                def _():
                    @pl.when(bt >= 1)
                    def _():
                        o_desc(bt - 1, nb).wait()
                    g_desc(bt + 1, nb).start()

                g_desc(bt, b).wait()
                select_scale(bt, b)
                o_desc(bt, b).start()
            return 0

        lax.fori_loop(0, n_batches // NBUF, outer, 0)
        o_desc(n_batches - 2, (n_batches - 2) % NBUF).wait()
        o_desc(n_batches - 1, (n_batches - 1) % NBUF).wait()

    return _emb


def kernel(x, table):
    R, S = x.shape
    V = table.shape[0]
    idx = x.reshape(-1).astype(jnp.int32)
    tab2 = table.reshape(V // 2, 2 * D)
    out = _make_emb(idx.shape[0])(idx, tab2)
    return out.reshape(R, S, D)
